# baseline (speedup 1.0000x reference)
import jax
import jax.numpy as jnp
from jax.experimental import pallas as pl
from jax.experimental.pallas import tpu as pltpu

_BS = 512


def _body(keep_ref, x_ref, mt_ref, o_ref):
    b = pl.program_id(0)
    j = pl.program_id(1)
    k = keep_ref[b]
    pos = j * _BS + jax.lax.broadcasted_iota(jnp.int32, (_BS, 1), 0)
    drop = pos >= k
    o_ref[0] = jnp.where(drop, mt_ref[...][None, :], x_ref[0])


def kernel(x, mask_token, keep_k):
    B, S, D = x.shape
    grid_spec = pltpu.PrefetchScalarGridSpec(
        num_scalar_prefetch=1,
        grid=(B, S // _BS),
        in_specs=[
            pl.BlockSpec((1, _BS, D), lambda b, j, k_ref: (b, j, 0)),
            pl.BlockSpec((D,), lambda b, j, k_ref: (0,)),
        ],
        out_specs=pl.BlockSpec((1, _BS, D), lambda b, j, k_ref: (b, j, 0)),
    )
    return pl.pallas_call(
        _body,
        grid_spec=grid_spec,
        out_shape=jax.ShapeDtypeStruct((B, S, D), x.dtype),
    )(keep_k, x, mask_token)
